# Initial kernel scaffold; baseline (speedup 1.0000x reference)
import jax
import jax.numpy as jnp
from jax.experimental import pallas as pl


def kernel(X, edge_index, edge_weight, edge_attr, q, W1, b1, W2, b2, W3, b3, WI_in, WA_in, WS_in, WI_out, WA_out, WS_out):
    raise NotImplementedError("write your pallas kernel here")


# speedup vs baseline: 13.0813x; 13.0813x over previous
import functools

import jax
import jax.numpy as jnp
from jax import lax
from jax.experimental import pallas as pl
from jax.experimental.pallas import tpu as pltpu
from jax.experimental.pallas import tpu_sc as plsc

_N = 10000
_E = 160000
_H = 32
_R = 32
_CUTOFF_UPPER = 5.0

_HH = 16
_D = 9 * _HH
_NP = 10112
_K = 80
_NTILES = 16
_NBN = 1000
_EB = 2000


def _silu(x):
    return x / (1.0 + jnp.exp(-x))


def _prenode_body(x_ref, xn_ref, a_ref, s_ref, i_ref):
    x = x_ref[...]
    ss = jnp.sum(x * x, axis=0)
    inv = 1.0 / (ss + 1.0)
    xn = x * inv[None]
    xn_ref[...] = xn
    tr = (xn[0] + xn[4] + xn[8]) * (1.0 / 3.0)
    i_ref[...] = tr
    for i in range(3):
        for j in range(3):
            p = i * 3 + j
            a = 0.5 * (xn[p] - xn[j * 3 + i])
            a_ref[p] = a
            if i == j:
                s_ref[p] = xn[p] - a - tr
            else:
                s_ref[p] = xn[p] - a


def _prenode(x9):
    spec9 = pl.BlockSpec((9, _NBN, _H), lambda n: (0, n, 0))
    spec2 = pl.BlockSpec((_NBN, _H), lambda n: (n, 0))
    f32 = jnp.float32
    return pl.pallas_call(
        _prenode_body,
        grid=(_N // _NBN,),
        in_specs=[spec9],
        out_specs=[spec9, spec9, spec9, spec2],
        out_shape=[
            jax.ShapeDtypeStruct((9, _N, _H), f32),
            jax.ShapeDtypeStruct((9, _N, _H), f32),
            jax.ShapeDtypeStruct((9, _N, _H), f32),
            jax.ShapeDtypeStruct((_N, _H), f32),
        ],
    )(x9)


def _edgemlp_body(ea_ref, ew_ref, w1_ref, b1_ref, w2_ref, b2_ref, w3_ref,
                  b3_ref, f_ref):
    f32 = jnp.float32
    h = _silu(jnp.dot(ea_ref[...], w1_ref[...].T, preferred_element_type=f32)
              + b1_ref[...])
    h = _silu(jnp.dot(h, w2_ref[...].T, preferred_element_type=f32)
              + b2_ref[...])
    h = _silu(jnp.dot(h, w3_ref[...].T, preferred_element_type=f32)
              + b3_ref[...])
    w = ew_ref[...]
    c = 0.5 * (jnp.cos(w * (jnp.pi / _CUTOFF_UPPER)) + 1.0)
    c = jnp.where(w < _CUTOFF_UPPER, c, 0.0)
    f_ref[...] = h * c


def _edgemlp(ea, ew, w1, b1, w2, b2, w3p, b3p):
    full = lambda shape: pl.BlockSpec(shape, lambda e: tuple(0 for _ in shape))
    return pl.pallas_call(
        _edgemlp_body,
        grid=(_E // _EB,),
        in_specs=[
            pl.BlockSpec((_EB, _R), lambda e: (e, 0)),
            pl.BlockSpec((_EB, 1), lambda e: (e, 0)),
            full((_H, _R)),
            full((1, _H)),
            full((2 * _H, _H)),
            full((1, 2 * _H)),
            full((3 * _H, 2 * _H)),
            full((1, 3 * _H)),
        ],
        out_specs=pl.BlockSpec((_EB, 3 * _H), lambda e: (e, 0)),
        out_shape=jax.ShapeDtypeStruct((_E, 3 * _H), jnp.float32),
    )(ea, ew, w1, b1, w2, b2, w3p, b3p)


def _sc_body(a_hbm, s_hbm, i_hbm, f_hbm, dst2_hbm, src_hbm, zer_hbm, y_hbm,
             idxd_v, idxs_v, rowsa_v, rowss_v, rowsi_v, fbuf_v, msg_v,
             yacc_sh, sem):
    c = lax.axis_index("c")
    t = lax.axis_index("s")
    npt = _NP // _NTILES
    ept = _E // _NTILES
    nchunks = ept // _K

    pltpu.sync_copy(zer_hbm, yacc_sh.at[pl.ds(t * npt, npt)])
    plsc.subcore_barrier()

    def chunk(ic, carry):
        base = t * ept + ic * _K
        pltpu.sync_copy(dst2_hbm.at[pl.ds(c * _E + base, _K)], idxd_v)
        pltpu.sync_copy(src_hbm.at[pl.ds(base, _K)], idxs_v)
        pltpu.sync_copy(f_hbm.at[pl.ds(c * _E + base, _K)], fbuf_v)
        ca = pltpu.async_copy(a_hbm.at[idxd_v], rowsa_v, sem)
        cs = pltpu.async_copy(s_hbm.at[idxd_v], rowss_v, sem)
        ci = pltpu.async_copy(i_hbm.at[idxd_v], rowsi_v, sem)
        ca.wait()
        cs.wait()
        ci.wait()

        def edge(e, ecarry):
            f0 = fbuf_v[e, pl.ds(0, 16)]
            f1 = fbuf_v[e, pl.ds(16, 16)]
            f2 = fbuf_v[e, pl.ds(32, 16)]
            fi = f0 * rowsi_v[e, pl.ds(0, 16)]
            for i in range(3):
                for j in range(3):
                    p = i * 3 + j
                    a = rowsa_v[e, pl.ds(p * 16, 16)]
                    sv = rowss_v[e, pl.ds(p * 16, 16)]
                    m = f1 * a + f2 * sv
                    if i == j:
                        m = m + fi
                    msg_v[e, pl.ds(p * 16, 16)] = m
            return ecarry

        lax.fori_loop(0, _K, edge, 0)
        pltpu.sync_copy(msg_v, yacc_sh.at[idxs_v], add=True)
        return carry

    lax.fori_loop(0, nchunks, chunk, 0)
    plsc.subcore_barrier()
    pltpu.sync_copy(yacc_sh.at[pl.ds(t * npt, npt)],
                    y_hbm.at[pl.ds(c * _NP + t * npt, npt)])


def _sc_scatter(a_tab, s_tab, i_tab, f_tab, dst2, src, zer):
    f32 = jnp.float32
    return pl.kernel(
        _sc_body,
        out_type=jax.ShapeDtypeStruct((2 * _NP, _D), f32),
        mesh=plsc.VectorSubcoreMesh(core_axis_name="c", subcore_axis_name="s"),
        compiler_params=pltpu.CompilerParams(use_tc_tiling_on_sc=False),
        scratch_types=[
            pltpu.VMEM((_K,), jnp.int32),
            pltpu.VMEM((_K,), jnp.int32),
            pltpu.VMEM((_K, _D), f32),
            pltpu.VMEM((_K, _D), f32),
            pltpu.VMEM((_K, _HH), f32),
            pltpu.VMEM((_K, 3 * _HH), f32),
            pltpu.VMEM((_K, _D), f32),
            pltpu.VMEM_SHARED((_NP, _D), f32),
            pltpu.SemaphoreType.DMA,
        ],
    )(a_tab, s_tab, i_tab, f_tab, dst2, src, zer)


def _postnode_body(xn_ref, y_ref, q_ref, wii_ref, wai_ref, wsi_ref,
                   wio_ref, wao_ref, wso_ref, o_ref):
    f32 = jnp.float32
    xn = xn_ref[...]
    y = [y_ref[p] for p in range(9)]

    def decompose(xs):
        tr = (xs[0] + xs[4] + xs[8]) * (1.0 / 3.0)
        aa, ss = [], []
        for i in range(3):
            for j in range(3):
                p = i * 3 + j
                a = 0.5 * (xs[p] - xs[j * 3 + i])
                aa.append(a)
                s = xs[p] - a - (tr if i == j else 0.0)
                ss.append(s)
        return tr, aa, ss

    def tensor_linear(xs, wi, wa, ws):
        tr, aa, ss = decompose(xs)
        iout = jnp.dot(tr, wi.T, preferred_element_type=f32)
        out = []
        for i in range(3):
            for j in range(3):
                p = i * 3 + j
                d = (jnp.dot(aa[p], wa.T, preferred_element_type=f32)
                     + jnp.dot(ss[p], ws.T, preferred_element_type=f32))
                if i == j:
                    d = d + iout
                out.append(d)
        return out

    def mat33(u, v):
        return [sum(u[i * 3 + k] * v[k * 3 + j] for k in range(3))
                for i in range(3) for j in range(3)]

    xn_l = [xn[p] for p in range(9)]
    xin = tensor_linear(xn_l, wii_ref[...], wai_ref[...], wsi_ref[...])
    bm = mat33(xin, y)
    am = mat33(y, xin)
    xnew = [am[p] + bm[p] for p in range(9)]
    ssq = sum(v * v for v in xnew)
    inv = 1.0 / (ssq + 1.0)
    xnn = [v * inv for v in xnew]
    dx = tensor_linear(xnn, wio_ref[...], wao_ref[...], wso_ref[...])
    dd = mat33(dx, dx)
    cf = 1.0 + 0.1 * q_ref[...]
    for p in range(9):
        o_ref[p] = xn_l[p] + (dx[p] + dd[p]) * cf


def _postnode(xn9, y9, q2, wii, wai, wsi, wio, wao, wso):
    spec9 = pl.BlockSpec((9, _NBN, _H), lambda n: (0, n, 0))
    specq = pl.BlockSpec((_NBN, 1), lambda n: (n, 0))
    specw = pl.BlockSpec((_H, _H), lambda n: (0, 0))
    return pl.pallas_call(
        _postnode_body,
        grid=(_N // _NBN,),
        in_specs=[spec9, spec9, specq, specw, specw, specw, specw, specw,
                  specw],
        out_specs=spec9,
        out_shape=jax.ShapeDtypeStruct((9, _N, _H), jnp.float32),
    )(xn9, y9, q2, wii, wai, wsi, wio, wao, wso)


@jax.jit
def kernel(X, edge_index, edge_weight, edge_attr, q, W1, b1, W2, b2, W3, b3,
           WI_in, WA_in, WS_in, WI_out, WA_out, WS_out):
    f32 = jnp.float32
    x9 = jnp.transpose(X, (2, 3, 0, 1)).reshape(9, _N, _H)
    xn9, a9, s9, itr = _prenode(x9)

    a_t = jnp.transpose(a9, (1, 0, 2))
    s_t = jnp.transpose(s9, (1, 0, 2))
    a_tab = jnp.concatenate([a_t[:, :, :_HH].reshape(_N, _D),
                             a_t[:, :, _HH:].reshape(_N, _D)], axis=0)
    s_tab = jnp.concatenate([s_t[:, :, :_HH].reshape(_N, _D),
                             s_t[:, :, _HH:].reshape(_N, _D)], axis=0)
    i_tab = jnp.concatenate([itr[:, :_HH], itr[:, _HH:]], axis=0)

    w3p = W3.reshape(_H, 3, 2 * _H).transpose(1, 0, 2).reshape(3 * _H, 2 * _H)
    b3p = b3.reshape(_H, 3).T.reshape(3 * _H)
    f = _edgemlp(edge_attr, edge_weight.reshape(_E, 1),
                 W1, b1.reshape(1, _H), W2, b2.reshape(1, 2 * _H),
                 w3p, b3p.reshape(1, 3 * _H))
    fr = f.reshape(_E, 3, 2, _HH)
    f_tab = jnp.concatenate([fr[:, :, 0].reshape(_E, 3 * _HH),
                             fr[:, :, 1].reshape(_E, 3 * _HH)], axis=0)

    src = edge_index[0]
    dst = edge_index[1]
    dst2 = jnp.concatenate([dst, dst + _N])
    zer = jnp.zeros((_NP // _NTILES, _D), f32)

    y2 = _sc_scatter(a_tab, s_tab, i_tab, f_tab, dst2, src, zer)

    y9 = (y2.reshape(2, _NP, 9, _HH)[:, :_N]
          .transpose(2, 1, 0, 3).reshape(9, _N, _H))
    o9 = _postnode(xn9, y9, q.reshape(_N, 1),
                   WI_in, WA_in, WS_in, WI_out, WA_out, WS_out)
    return o9.transpose(1, 2, 0).reshape(_N, _H, 3, 3)


# speedup vs baseline: 16.8254x; 1.2862x over previous
import functools

import jax
import jax.numpy as jnp
from jax import lax
from jax.experimental import pallas as pl
from jax.experimental.pallas import tpu as pltpu
from jax.experimental.pallas import tpu_sc as plsc

_N = 10000
_E = 160000
_H = 32
_R = 32
_CUTOFF_UPPER = 5.0

_HH = 16
_D = 9 * _HH
_NP = 10112
_K = 80
_NTILES = 16
_NBN = 1000
_EB = 2000


def _silu(x):
    return x / (1.0 + jnp.exp(-x))


def _prenode_body(x_ref, pin_ref, xn_ref, a_ref, s_ref, i_ref):
    f32 = jnp.float32
    xp = jnp.dot(x_ref[...], pin_ref[...], preferred_element_type=f32)
    xs = [xp[:, p * _H:(p + 1) * _H] for p in range(9)]
    ssq = sum(v * v for v in xs)
    inv = 1.0 / (ssq + 1.0)
    xn = [v * inv for v in xs]
    xn_ref[...] = jnp.concatenate(xn, axis=1)
    tr = (xn[0] + xn[4] + xn[8]) * (1.0 / 3.0)
    i_ref[0] = tr[:, :_HH]
    i_ref[1] = tr[:, _HH:]
    aa, ss = [], []
    for i in range(3):
        for j in range(3):
            p = i * 3 + j
            a = 0.5 * (xn[p] - xn[j * 3 + i])
            aa.append(a)
            ss.append(xn[p] - a - (tr if i == j else 0.0))
    a_ref[0] = jnp.concatenate([v[:, :_HH] for v in aa], axis=1)
    a_ref[1] = jnp.concatenate([v[:, _HH:] for v in aa], axis=1)
    s_ref[0] = jnp.concatenate([v[:, :_HH] for v in ss], axis=1)
    s_ref[1] = jnp.concatenate([v[:, _HH:] for v in ss], axis=1)


def _prenode(x288, pin):
    f32 = jnp.float32
    spec_pm = pl.BlockSpec((_NBN, 288), lambda n: (n, 0))
    spec_tab = pl.BlockSpec((2, _NBN, _D), lambda n: (0, n, 0))
    return pl.pallas_call(
        _prenode_body,
        grid=(_N // _NBN,),
        in_specs=[spec_pm, pl.BlockSpec((288, 288), lambda n: (0, 0))],
        out_specs=[spec_pm, spec_tab, spec_tab,
                   pl.BlockSpec((2, _NBN, _HH), lambda n: (0, n, 0))],
        out_shape=[
            jax.ShapeDtypeStruct((_N, 288), f32),
            jax.ShapeDtypeStruct((2, _N, _D), f32),
            jax.ShapeDtypeStruct((2, _N, _D), f32),
            jax.ShapeDtypeStruct((2, _N, _HH), f32),
        ],
    )(x288, pin)


def _edgemlp_body(ea_ref, ew_ref, w1_ref, b1_ref, w2_ref, b2_ref, w3_ref,
                  b3_ref, f_ref):
    f32 = jnp.float32
    h = _silu(jnp.dot(ea_ref[...], w1_ref[...].T, preferred_element_type=f32)
              + b1_ref[...])
    h = _silu(jnp.dot(h, w2_ref[...].T, preferred_element_type=f32)
              + b2_ref[...])
    h = _silu(jnp.dot(h, w3_ref[...].T, preferred_element_type=f32)
              + b3_ref[...])
    w = ew_ref[...]
    c = 0.5 * (jnp.cos(w * (jnp.pi / _CUTOFF_UPPER)) + 1.0)
    c = jnp.where(w < _CUTOFF_UPPER, c, 0.0)
    h = h * c
    f_ref[0] = jnp.concatenate(
        [h[:, k * _H:k * _H + _HH] for k in range(3)], axis=1)
    f_ref[1] = jnp.concatenate(
        [h[:, k * _H + _HH:(k + 1) * _H] for k in range(3)], axis=1)


def _edgemlp(ea, ew, w1, b1, w2, b2, w3p, b3p):
    full = lambda shape: pl.BlockSpec(shape, lambda e: tuple(0 for _ in shape))
    return pl.pallas_call(
        _edgemlp_body,
        grid=(_E // _EB,),
        in_specs=[
            pl.BlockSpec((_EB, _R), lambda e: (e, 0)),
            pl.BlockSpec((_EB, 1), lambda e: (e, 0)),
            full((_H, _R)),
            full((1, _H)),
            full((2 * _H, _H)),
            full((1, 2 * _H)),
            full((3 * _H, 2 * _H)),
            full((1, 3 * _H)),
        ],
        out_specs=pl.BlockSpec((2, _EB, 3 * _HH), lambda e: (0, e, 0)),
        out_shape=jax.ShapeDtypeStruct((2, _E, 3 * _HH), jnp.float32),
    )(ea, ew, w1, b1, w2, b2, w3p, b3p)


def _sc_body(a_hbm, s_hbm, i_hbm, f_hbm, dst2_hbm, src_hbm, zer_hbm, y_hbm,
             idxd_v, idxs_v, rowsa_v, rowss_v, rowsi_v, fbuf_v, msg_v,
             yacc_sh, sem):
    c = lax.axis_index("c")
    t = lax.axis_index("s")
    npt = _NP // _NTILES
    ept = _E // _NTILES
    nchunks = ept // _K

    pltpu.sync_copy(zer_hbm, yacc_sh.at[pl.ds(t * npt, npt)])
    plsc.subcore_barrier()

    def chunk(ic, carry):
        base = t * ept + ic * _K
        pltpu.sync_copy(dst2_hbm.at[pl.ds(c * _E + base, _K)], idxd_v)
        pltpu.sync_copy(src_hbm.at[pl.ds(base, _K)], idxs_v)
        pltpu.sync_copy(f_hbm.at[pl.ds(c * _E + base, _K)], fbuf_v)
        ca = pltpu.async_copy(a_hbm.at[idxd_v], rowsa_v, sem)
        cs = pltpu.async_copy(s_hbm.at[idxd_v], rowss_v, sem)
        ci = pltpu.async_copy(i_hbm.at[idxd_v], rowsi_v, sem)
        ca.wait()
        cs.wait()
        ci.wait()

        def edge(e, ecarry):
            f0 = fbuf_v[e, pl.ds(0, 16)]
            f1 = fbuf_v[e, pl.ds(16, 16)]
            f2 = fbuf_v[e, pl.ds(32, 16)]
            fi = f0 * rowsi_v[e, pl.ds(0, 16)]
            for i in range(3):
                for j in range(3):
                    p = i * 3 + j
                    a = rowsa_v[e, pl.ds(p * 16, 16)]
                    sv = rowss_v[e, pl.ds(p * 16, 16)]
                    m = f1 * a + f2 * sv
                    if i == j:
                        m = m + fi
                    msg_v[e, pl.ds(p * 16, 16)] = m
            return ecarry

        lax.fori_loop(0, _K, edge, 0)
        pltpu.sync_copy(msg_v, yacc_sh.at[idxs_v], add=True)
        return carry

    lax.fori_loop(0, nchunks, chunk, 0)
    plsc.subcore_barrier()
    pltpu.sync_copy(yacc_sh.at[pl.ds(t * npt, npt)],
                    y_hbm.at[pl.ds(c * _NP + t * npt, npt)])


def _sc_scatter(a_tab, s_tab, i_tab, f_tab, dst2, src, zer):
    f32 = jnp.float32
    return pl.kernel(
        _sc_body,
        out_type=jax.ShapeDtypeStruct((2 * _NP, _D), f32),
        mesh=plsc.VectorSubcoreMesh(core_axis_name="c", subcore_axis_name="s"),
        compiler_params=pltpu.CompilerParams(use_tc_tiling_on_sc=False),
        scratch_types=[
            pltpu.VMEM((_K,), jnp.int32),
            pltpu.VMEM((_K,), jnp.int32),
            pltpu.VMEM((_K, _D), f32),
            pltpu.VMEM((_K, _D), f32),
            pltpu.VMEM((_K, _HH), f32),
            pltpu.VMEM((_K, 3 * _HH), f32),
            pltpu.VMEM((_K, _D), f32),
            pltpu.VMEM_SHARED((_NP, _D), f32),
            pltpu.SemaphoreType.DMA,
        ],
    )(a_tab, s_tab, i_tab, f_tab, dst2, src, zer)


def _postnode_body(xn_ref, y_ref, q_ref, wii_ref, wai_ref, wsi_ref,
                   wio_ref, wao_ref, wso_ref, pout_ref, o_ref):
    f32 = jnp.float32
    xnp = xn_ref[...]
    y0 = y_ref[0]
    y1 = y_ref[1]
    y = [jnp.concatenate([y0[:, p * _HH:(p + 1) * _HH],
                          y1[:, p * _HH:(p + 1) * _HH]], axis=1)
         for p in range(9)]

    def decompose(xs):
        tr = (xs[0] + xs[4] + xs[8]) * (1.0 / 3.0)
        aa, ss = [], []
        for i in range(3):
            for j in range(3):
                p = i * 3 + j
                a = 0.5 * (xs[p] - xs[j * 3 + i])
                aa.append(a)
                s = xs[p] - a - (tr if i == j else 0.0)
                ss.append(s)
        return tr, aa, ss

    def tensor_linear(xs, wi, wa, ws):
        tr, aa, ss = decompose(xs)
        iout = jnp.dot(tr, wi.T, preferred_element_type=f32)
        out = []
        for i in range(3):
            for j in range(3):
                p = i * 3 + j
                d = (jnp.dot(aa[p], wa.T, preferred_element_type=f32)
                     + jnp.dot(ss[p], ws.T, preferred_element_type=f32))
                if i == j:
                    d = d + iout
                out.append(d)
        return out

    def mat33(u, v):
        return [sum(u[i * 3 + k] * v[k * 3 + j] for k in range(3))
                for i in range(3) for j in range(3)]

    xn_l = [xnp[:, p * _H:(p + 1) * _H] for p in range(9)]
    xin = tensor_linear(xn_l, wii_ref[...], wai_ref[...], wsi_ref[...])
    bm = mat33(xin, y)
    am = mat33(y, xin)
    xnew = [am[p] + bm[p] for p in range(9)]
    ssq = sum(v * v for v in xnew)
    inv = 1.0 / (ssq + 1.0)
    xnn = [v * inv for v in xnew]
    dx = tensor_linear(xnn, wio_ref[...], wao_ref[...], wso_ref[...])
    dd = mat33(dx, dx)
    cf = 1.0 + 0.1 * q_ref[...]
    o_pm = jnp.concatenate(
        [xn_l[p] + (dx[p] + dd[p]) * cf for p in range(9)], axis=1)
    o_ref[...] = jnp.dot(o_pm, pout_ref[...], preferred_element_type=f32)


def _postnode(xn_pm, y2, q2, wii, wai, wsi, wio, wao, wso, pout):
    spec_pm = pl.BlockSpec((_NBN, 288), lambda n: (n, 0))
    specy = pl.BlockSpec((2, _NBN, _D), lambda n: (0, n, 0))
    specq = pl.BlockSpec((_NBN, 1), lambda n: (n, 0))
    specw = pl.BlockSpec((_H, _H), lambda n: (0, 0))
    specp = pl.BlockSpec((288, 288), lambda n: (0, 0))
    return pl.pallas_call(
        _postnode_body,
        grid=(_N // _NBN,),
        in_specs=[spec_pm, specy, specq, specw, specw, specw, specw, specw,
                  specw, specp],
        out_specs=spec_pm,
        out_shape=jax.ShapeDtypeStruct((_N, 288), jnp.float32),
    )(xn_pm, y2, q2, wii, wai, wsi, wio, wao, wso, pout)


@jax.jit
def kernel(X, edge_index, edge_weight, edge_attr, q, W1, b1, W2, b2, W3, b3,
           WI_in, WA_in, WS_in, WI_out, WA_out, WS_out):
    f32 = jnp.float32
    idx = jnp.arange(288)
    pin = jax.nn.one_hot((idx % 9) * _H + idx // 9, 288, dtype=f32)
    pout = jax.nn.one_hot((idx % _H) * 9 + idx // _H, 288, dtype=f32)

    xn_pm, a_tab, s_tab, i_tab = _prenode(X.reshape(_N, 288), pin)

    w3p = W3.reshape(_H, 3, 2 * _H).transpose(1, 0, 2).reshape(3 * _H, 2 * _H)
    b3p = b3.reshape(_H, 3).T.reshape(3 * _H)
    f_tab = _edgemlp(edge_attr, edge_weight.reshape(_E, 1),
                     W1, b1.reshape(1, _H), W2, b2.reshape(1, 2 * _H),
                     w3p, b3p.reshape(1, 3 * _H))

    src = edge_index[0]
    dst = edge_index[1]
    dst2 = jnp.concatenate([dst, dst + _N])
    zer = jnp.zeros((_NP // _NTILES, _D), f32)

    y2 = _sc_scatter(a_tab.reshape(2 * _N, _D), s_tab.reshape(2 * _N, _D),
                     i_tab.reshape(2 * _N, _HH), f_tab.reshape(2 * _E, 3 * _HH),
                     dst2, src, zer)

    o = _postnode(xn_pm, y2.reshape(2, _NP, _D), q.reshape(_N, 1),
                  WI_in, WA_in, WS_in, WI_out, WA_out, WS_out, pout)
    return o.reshape(_N, _H, 3, 3)
